# Initial kernel scaffold; baseline (speedup 1.0000x reference)
import jax
import jax.numpy as jnp
from jax.experimental import pallas as pl


def kernel(hidden_states, params, attention_mask):
    raise NotImplementedError("write your pallas kernel here")


# speedup vs baseline: 5.9424x; 5.9424x over previous
import functools

import jax
import jax.numpy as jnp
from jax import lax
from jax.experimental import pallas as pl

_INV_SQRT2 = 0.7071067811865476
_BAND = 32
_MAX_SPAN = 30
_TOP_LAMBDA = 0.4
_EPS = 1e-5
_NEG = -1e9


def _mlp_body(x_ref, w_ref, b_ref, g_ref, bt_ref, o_ref):
    y = jnp.dot(x_ref[:], w_ref[:], preferred_element_type=jnp.float32) + b_ref[:]
    y = 0.5 * y * (1.0 + lax.erf(y * _INV_SQRT2))
    mu = jnp.mean(y, axis=-1, keepdims=True)
    yc = y - mu
    var = jnp.mean(yc * yc, axis=-1, keepdims=True)
    o_ref[:] = yc * lax.rsqrt(var + _EPS) * g_ref[:] + bt_ref[:]


def _band_body(sm_ref, emlo_ref, emhi_ref, w_ref, sb_ref, msw_ref, mew_ref,
               o_ref, *, rb, s):
    r = pl.program_id(0)
    sm_b = sm_ref[:]
    em_win = jnp.concatenate([emlo_ref[:], emhi_ref[:_BAND]], axis=0)
    temp = jnp.dot(sm_b, w_ref[:], preferred_element_type=jnp.float32) + sb_ref[:]
    sml = jnp.dot(sm_b, msw_ref[:], preferred_element_type=jnp.float32)
    eml = lax.dot_general(mew_ref[:], em_win, (((1,), (1,)), ((), ())),
                          preferred_element_type=jnp.float32)
    m = lax.dot_general(temp, em_win, (((1,), (1,)), ((), ())),
                        preferred_element_type=jnp.float32) + eml
    w = rb + _BAND
    cio = lax.broadcasted_iota(jnp.int32, (rb, w), 1)
    rio = lax.broadcasted_iota(jnp.int32, (rb, w), 0)
    dio = lax.broadcasted_iota(jnp.int32, (rb, _BAND), 1)

    def dstep(d, acc):
        v = jnp.sum(jnp.where(cio == rio + d, m, 0.0), axis=1, keepdims=True)
        return acc + v * (dio == d).astype(jnp.float32)

    acc = lax.fori_loop(0, _BAND, dstep, jnp.zeros((rb, _BAND), jnp.float32))
    band = acc + sml
    rg = r * rb + lax.broadcasted_iota(jnp.int32, (rb, _BAND), 0)
    ok = (dio < _MAX_SPAN) & (rg + dio < s)
    o_ref[:] = jnp.where(ok, jnp.clip(band, -1e4, 1e4), _NEG)


def _select_body(band_ref, o_ref, *, s, maxk, kpad):
    b = band_ref[:]
    valid = b > -1e8
    mn = jnp.min(jnp.where(valid, b, 1e30))
    mx = jnp.max(b)

    def bis(_, carry):
        lo, hi = carry
        mid = 0.5 * (lo + hi)
        cnt = jnp.sum((b > mid).astype(jnp.float32))
        geq = cnt >= float(maxk)
        return (jnp.where(geq, mid, lo), jnp.where(geq, hi, mid))

    lo, hi = lax.fori_loop(0, 46, bis, (mn - 1.0, mx))
    sel_hi = b > hi
    sel_tie = (b > lo) & jnp.logical_not(sel_hi)
    m_cnt = jnp.sum(sel_hi.astype(jnp.float32))
    need = float(maxk) - m_cnt

    bio_r = lax.broadcasted_iota(jnp.int32, (_BAND, _BAND), 0)
    bio_c = lax.broadcasted_iota(jnp.int32, (_BAND, _BAND), 1)
    u32 = (bio_r < bio_c).astype(jnp.float32)
    ones_col = jnp.ones((_BAND, 1), jnp.float32)

    tio_r = lax.broadcasted_iota(jnp.int32, (s, s), 0)
    tio_c = lax.broadcasted_iota(jnp.int32, (s, s), 1)
    tri = (tio_c < tio_r).astype(jnp.float32)

    tiec = sel_tie.astype(jnp.float32)
    c1_tie = jnp.dot(tiec, u32, preferred_element_type=jnp.float32)
    cnt_tie_col = jnp.dot(tiec, ones_col, preferred_element_type=jnp.float32)
    offs_tie_col = jnp.dot(tri, cnt_tie_col, preferred_element_type=jnp.float32)
    accept = need - offs_tie_col
    sel = sel_hi | (sel_tie & (c1_tie < accept))

    selc = sel.astype(jnp.float32)
    c1 = jnp.dot(selc, u32, preferred_element_type=jnp.float32)
    cnt_row = lax.dot_general(jnp.ones((1, _BAND), jnp.float32), selc,
                              (((1,), (1,)), ((), ())),
                              preferred_element_type=jnp.float32)
    offs_row = lax.dot_general(cnt_row, tri, (((1,), (1,)), ((), ())),
                               preferred_element_type=jnp.float32)

    sio_r = lax.broadcasted_iota(jnp.int32, (s, _BAND), 0)
    sio_c = lax.broadcasted_iota(jnp.int32, (s, _BAND), 1)
    code = (sio_r * _BAND + sio_c).astype(jnp.float32)
    iok = lax.broadcasted_iota(jnp.int32, (kpad, s), 0).astype(jnp.float32)

    def tstep(t, acc):
        tf = t.astype(jnp.float32)
        ind = (sel & (c1 == tf)).astype(jnp.float32)
        vcode = jnp.sum(ind * code, axis=1, keepdims=True)
        vml = jnp.sum(ind * b, axis=1, keepdims=True)
        vals = jnp.concatenate([vcode, vml], axis=1)
        m2 = ((iok == offs_row + tf) & (cnt_row > tf))
        return acc + jnp.dot(m2.astype(jnp.float32), vals,
                             precision=lax.Precision.HIGHEST,
                             preferred_element_type=jnp.float32)

    o_ref[:] = lax.fori_loop(0, _BAND, tstep, jnp.zeros((kpad, 2), jnp.float32))


def _gather_body(sidx_ref, eidx_ref, sc_ref, ec_ref, ts_ref, te_ref, *, kb, kpad):
    k = pl.program_id(0)

    @pl.when(k == 0)
    def _():
        ts_ref[:] = jnp.zeros_like(ts_ref)
        te_ref[:] = jnp.zeros_like(te_ref)

    base = (k * kb + lax.broadcasted_iota(jnp.int32, (kpad, kb), 1)).astype(jnp.float32)
    ohs = (sidx_ref[0] == base).astype(jnp.float32)
    ohe = (eidx_ref[0] == base).astype(jnp.float32)
    ts_ref[:] += jnp.dot(ohs, sc_ref[:], preferred_element_type=jnp.float32)
    te_ref[:] += jnp.dot(ohe, ec_ref[:], preferred_element_type=jnp.float32)


def _pq_body(ts_ref, te_ref, ass_ref, aes_ref, ase_ref, aee_ref, bp_ref, bq_ref,
             p_ref, q_ref):
    ts = ts_ref[:]
    te = te_ref[:]
    p_ref[:] = (jnp.dot(ts, ass_ref[:], preferred_element_type=jnp.float32)
                + jnp.dot(te, aes_ref[:], preferred_element_type=jnp.float32)
                + bp_ref[:])
    q_ref[:] = (jnp.dot(ts, ase_ref[:], preferred_element_type=jnp.float32)
                + jnp.dot(te, aee_ref[:], preferred_element_type=jnp.float32)
                + bq_ref[:])


def _final_body(p_ref, q_ref, ts_ref, te_ref, mlc_ref, mlr_ref, o_ref, *, fb, kpad):
    i = pl.program_id(0)
    nt = (((1,), (1,)), ((), ()))
    c = (lax.dot_general(p_ref[:], ts_ref[:], nt, preferred_element_type=jnp.float32)
         + lax.dot_general(q_ref[:], te_ref[:], nt, preferred_element_type=jnp.float32))
    x = c + mlc_ref[:] + mlr_ref[:]
    rg = i * fb + lax.broadcasted_iota(jnp.int32, (fb, kpad), 0)
    cg = lax.broadcasted_iota(jnp.int32, (fb, kpad), 1)
    o_ref[:] = jnp.where(cg < rg, jnp.clip(x, -1e4, 1e4),
                         jnp.clip(x - 1e4, -1e4, 1e4))


def kernel(hidden_states, params, attention_mask):
    p = params
    x = hidden_states[0]
    s, h = x.shape
    f = p['sm_W'].shape[1]
    maxk = int(s * _TOP_LAMBDA)
    kpad = ((maxk + 31) // 32) * 32
    rb = 256 if s % 256 == 0 else s
    nrb = s // rb

    row = lambda v: v.reshape(1, -1)

    def mlp(pre):
        return pl.pallas_call(
            _mlp_body,
            grid=(nrb,),
            in_specs=[
                pl.BlockSpec((rb, h), lambda i: (i, 0)),
                pl.BlockSpec((h, f), lambda i: (0, 0)),
                pl.BlockSpec((1, f), lambda i: (0, 0)),
                pl.BlockSpec((1, f), lambda i: (0, 0)),
                pl.BlockSpec((1, f), lambda i: (0, 0)),
            ],
            out_specs=pl.BlockSpec((rb, f), lambda i: (i, 0)),
            out_shape=jax.ShapeDtypeStruct((s, f), jnp.float32),
        )(x, p[pre + '_W'], row(p[pre + '_b']), row(p[pre + '_g']),
          row(p[pre + '_beta']))

    sm = mlp('sm')
    em = mlp('em')
    scv = mlp('sc')
    ecv = mlp('ec')

    last = nrb - 1
    band = pl.pallas_call(
        functools.partial(_band_body, rb=rb, s=s),
        grid=(nrb,),
        in_specs=[
            pl.BlockSpec((rb, f), lambda i: (i, 0)),
            pl.BlockSpec((rb, f), lambda i: (i, 0)),
            pl.BlockSpec((rb, f), lambda i: (jnp.minimum(i + 1, last), 0)),
            pl.BlockSpec((f, f), lambda i: (0, 0)),
            pl.BlockSpec((1, f), lambda i: (0, 0)),
            pl.BlockSpec((f, 1), lambda i: (0, 0)),
            pl.BlockSpec((1, f), lambda i: (0, 0)),
        ],
        out_specs=pl.BlockSpec((rb, _BAND), lambda i: (i, 0)),
        out_shape=jax.ShapeDtypeStruct((s, _BAND), jnp.float32),
    )(sm, em, em, p['s2e_W'], row(p['s2e_b']), p['ms_W'], row(p['me_W'][:, 0]))

    sel_out = pl.pallas_call(
        functools.partial(_select_body, s=s, maxk=maxk, kpad=kpad),
        in_specs=[pl.BlockSpec((s, _BAND), lambda: (0, 0))],
        out_specs=pl.BlockSpec((kpad, 2), lambda: (0, 0)),
        out_shape=jax.ShapeDtypeStruct((kpad, 2), jnp.float32),
    )(band)

    code = jnp.round(sel_out[:, 0:1])
    startf = jnp.floor(code / _BAND)
    endf = startf + (code - _BAND * startf)
    mlc = sel_out[:, 1:2]
    mlr = mlc.reshape(1, kpad)

    kb = rb
    ts, te = pl.pallas_call(
        functools.partial(_gather_body, kb=kb, kpad=kpad),
        grid=(nrb,),
        in_specs=[
            pl.BlockSpec((1, kpad, 1), lambda i: (0, 0, 0)),
            pl.BlockSpec((1, kpad, 1), lambda i: (0, 0, 0)),
            pl.BlockSpec((kb, f), lambda i: (i, 0)),
            pl.BlockSpec((kb, f), lambda i: (i, 0)),
        ],
        out_specs=[
            pl.BlockSpec((kpad, f), lambda i: (0, 0)),
            pl.BlockSpec((kpad, f), lambda i: (0, 0)),
        ],
        out_shape=[
            jax.ShapeDtypeStruct((kpad, f), jnp.float32),
            jax.ShapeDtypeStruct((kpad, f), jnp.float32),
        ],
    )(startf.reshape(1, kpad, 1), endf.reshape(1, kpad, 1), scv, ecv)

    cb = 256 if f % 256 == 0 else f
    ncb = f // cb
    bp = row(p['a_s2s_b'] + p['a_e2s_b'])
    bq = row(p['a_s2e_b'] + p['a_e2e_b'])
    pm, qm = pl.pallas_call(
        _pq_body,
        grid=(ncb,),
        in_specs=[
            pl.BlockSpec((kpad, f), lambda j: (0, 0)),
            pl.BlockSpec((kpad, f), lambda j: (0, 0)),
            pl.BlockSpec((f, cb), lambda j: (0, j)),
            pl.BlockSpec((f, cb), lambda j: (0, j)),
            pl.BlockSpec((f, cb), lambda j: (0, j)),
            pl.BlockSpec((f, cb), lambda j: (0, j)),
            pl.BlockSpec((1, cb), lambda j: (0, j)),
            pl.BlockSpec((1, cb), lambda j: (0, j)),
        ],
        out_specs=[
            pl.BlockSpec((kpad, cb), lambda j: (0, j)),
            pl.BlockSpec((kpad, cb), lambda j: (0, j)),
        ],
        out_shape=[
            jax.ShapeDtypeStruct((kpad, f), jnp.float32),
            jax.ShapeDtypeStruct((kpad, f), jnp.float32),
        ],
    )(ts, te, p['a_s2s_W'], p['a_e2s_W'], p['a_s2e_W'], p['a_e2e_W'], bp, bq)

    fb = kpad // 4 if (kpad // 4) % 8 == 0 else kpad
    nfb = kpad // fb
    out = pl.pallas_call(
        functools.partial(_final_body, fb=fb, kpad=kpad),
        grid=(nfb,),
        in_specs=[
            pl.BlockSpec((fb, f), lambda i: (i, 0)),
            pl.BlockSpec((fb, f), lambda i: (i, 0)),
            pl.BlockSpec((kpad, f), lambda i: (0, 0)),
            pl.BlockSpec((kpad, f), lambda i: (0, 0)),
            pl.BlockSpec((fb, 1), lambda i: (i, 0)),
            pl.BlockSpec((1, kpad), lambda i: (0, 0)),
        ],
        out_specs=pl.BlockSpec((fb, kpad), lambda i: (i, 0)),
        out_shape=jax.ShapeDtypeStruct((kpad, kpad), jnp.float32),
    )(pm, qm, ts, te, mlc, mlr)

    return out[None, :maxk, :maxk]


# speedup vs baseline: 9.2215x; 1.5518x over previous
import functools

import jax
import jax.numpy as jnp
from jax import lax
from jax.experimental import pallas as pl
from jax.experimental.pallas import tpu as pltpu, tpu_sc as plsc

_INV_SQRT2 = 0.7071067811865476
_BAND = 32
_MAX_SPAN = 30
_TOP_LAMBDA = 0.4
_EPS = 1e-5
_NEG = -1e9


def _mlp_body(x_ref, w_ref, b_ref, g_ref, bt_ref, o_ref):
    y = jnp.dot(x_ref[:], w_ref[:], preferred_element_type=jnp.float32) + b_ref[:]
    y = 0.5 * y * (1.0 + lax.erf(y * _INV_SQRT2))
    mu = jnp.mean(y, axis=-1, keepdims=True)
    yc = y - mu
    var = jnp.mean(yc * yc, axis=-1, keepdims=True)
    o_ref[:] = yc * lax.rsqrt(var + _EPS) * g_ref[:] + bt_ref[:]


def _band_body(sm_ref, emlo_ref, emhi_ref, w_ref, sb_ref, msw_ref, mew_ref,
               o_ref, *, rb, s):
    r = pl.program_id(0)
    sm_b = sm_ref[:]
    em_win = jnp.concatenate([emlo_ref[:], emhi_ref[:_BAND]], axis=0)
    temp = jnp.dot(sm_b, w_ref[:], preferred_element_type=jnp.float32) + sb_ref[:]
    sml = jnp.dot(sm_b, msw_ref[:], preferred_element_type=jnp.float32)
    eml = lax.dot_general(mew_ref[:], em_win, (((1,), (1,)), ((), ())),
                          preferred_element_type=jnp.float32)
    m = lax.dot_general(temp, em_win, (((1,), (1,)), ((), ())),
                        preferred_element_type=jnp.float32) + eml
    w = rb + _BAND
    cio = lax.broadcasted_iota(jnp.int32, (rb, w), 1)
    rio = lax.broadcasted_iota(jnp.int32, (rb, w), 0)
    dio = lax.broadcasted_iota(jnp.int32, (rb, _BAND), 1)

    def dstep(d, acc):
        v = jnp.sum(jnp.where(cio == rio + d, m, 0.0), axis=1, keepdims=True)
        return acc + v * (dio == d).astype(jnp.float32)

    acc = lax.fori_loop(0, _BAND, dstep, jnp.zeros((rb, _BAND), jnp.float32))
    band = acc + sml
    rg = r * rb + lax.broadcasted_iota(jnp.int32, (rb, _BAND), 0)
    ok = (dio < _MAX_SPAN) & (rg + dio < s)
    o_ref[:] = jnp.where(ok, jnp.clip(band, -1e4, 1e4), _NEG)


def _select_body(band_ref, o_ref, *, s, maxk, kpad):
    b = band_ref[:]
    valid = b > -1e8
    mn = jnp.min(jnp.where(valid, b, 1e30))
    mx = jnp.max(b)

    def bis(_, carry):
        lo, hi = carry
        mid = 0.5 * (lo + hi)
        cnt = jnp.sum((b > mid).astype(jnp.float32))
        geq = cnt >= float(maxk)
        return (jnp.where(geq, mid, lo), jnp.where(geq, hi, mid))

    lo, hi = lax.fori_loop(0, 46, bis, (mn - 1.0, mx))
    sel_hi = b > hi
    sel_tie = (b > lo) & jnp.logical_not(sel_hi)
    m_cnt = jnp.sum(sel_hi.astype(jnp.float32))
    need = float(maxk) - m_cnt

    bio_r = lax.broadcasted_iota(jnp.int32, (_BAND, _BAND), 0)
    bio_c = lax.broadcasted_iota(jnp.int32, (_BAND, _BAND), 1)
    u32 = (bio_r < bio_c).astype(jnp.float32)
    ones_col = jnp.ones((_BAND, 1), jnp.float32)

    tio_r = lax.broadcasted_iota(jnp.int32, (s, s), 0)
    tio_c = lax.broadcasted_iota(jnp.int32, (s, s), 1)
    tri = (tio_c < tio_r).astype(jnp.float32)

    tiec = sel_tie.astype(jnp.float32)
    c1_tie = jnp.dot(tiec, u32, preferred_element_type=jnp.float32)
    cnt_tie_col = jnp.dot(tiec, ones_col, preferred_element_type=jnp.float32)
    offs_tie_col = jnp.dot(tri, cnt_tie_col, preferred_element_type=jnp.float32)
    accept = need - offs_tie_col
    sel = sel_hi | (sel_tie & (c1_tie < accept))

    selc = sel.astype(jnp.float32)
    c1 = jnp.dot(selc, u32, preferred_element_type=jnp.float32)
    cnt_row = lax.dot_general(jnp.ones((1, _BAND), jnp.float32), selc,
                              (((1,), (1,)), ((), ())),
                              preferred_element_type=jnp.float32)
    offs_row = lax.dot_general(cnt_row, tri, (((1,), (1,)), ((), ())),
                               preferred_element_type=jnp.float32)

    sio_r = lax.broadcasted_iota(jnp.int32, (s, _BAND), 0)
    sio_c = lax.broadcasted_iota(jnp.int32, (s, _BAND), 1)
    code = (sio_r * _BAND + sio_c).astype(jnp.float32)
    iok = lax.broadcasted_iota(jnp.int32, (kpad, s), 0).astype(jnp.float32)

    vio = lax.broadcasted_iota(jnp.int32, (s, 2 * _BAND), 1)

    def vstep(t, v_acc):
        tf = t.astype(jnp.float32)
        ind = (sel & (c1 == tf)).astype(jnp.float32)
        vcode = jnp.sum(ind * code, axis=1, keepdims=True)
        vml = jnp.sum(ind * b, axis=1, keepdims=True)
        return (v_acc + vcode * (vio == 2 * t).astype(jnp.float32)
                + vml * (vio == 2 * t + 1).astype(jnp.float32))

    v = lax.fori_loop(0, _BAND, vstep, jnp.zeros((s, 2 * _BAND), jnp.float32))
    d0 = (iok == offs_row).astype(jnp.float32)
    contrib = jnp.dot(d0, v, precision=lax.Precision.HIGHEST,
                      preferred_element_type=jnp.float32)
    acc = contrib[:, 0:2]
    for t in range(1, _BAND):
        acc = acc + jnp.concatenate(
            [jnp.zeros((t, 2), jnp.float32), contrib[:kpad - t, 2 * t:2 * t + 2]],
            axis=0)
    o_ref[:] = acc


def _sc_gather_body(sidx_ref, eidx_ref, sc_ref, ec_ref, ts_ref, te_ref,
                    sidx_a, sidx_b, eidx_a, eidx_b, r16, sem, *, nw2):
    cid = lax.axis_index("c")
    sid = lax.axis_index("s")
    wid = sid * 2 + cid

    @pl.when(wid < nw2)
    def _():
        base = pl.multiple_of(wid * 32, 32)
        pltpu.sync_copy(sidx_ref.at[pl.ds(base, 16)], sidx_a)
        pltpu.sync_copy(sidx_ref.at[pl.ds(base + 16, 16)], sidx_b)
        pltpu.sync_copy(eidx_ref.at[pl.ds(base, 16)], eidx_a)
        pltpu.sync_copy(eidx_ref.at[pl.ds(base + 16, 16)], eidx_b)

        def move(tab, idxref, outhbm, row0):
            cp = pltpu.make_async_copy(tab.at[idxref], r16, sem)
            cp.start()
            cp.wait()
            pltpu.sync_copy(r16, outhbm.at[pl.ds(row0, 16)])

        move(sc_ref, sidx_a, ts_ref, base)
        move(sc_ref, sidx_b, ts_ref, base + 16)
        move(ec_ref, eidx_a, te_ref, base)
        move(ec_ref, eidx_b, te_ref, base + 16)


def _pq_body(ts_ref, te_ref, ass_ref, aes_ref, ase_ref, aee_ref, bp_ref, bq_ref,
             p_ref, q_ref):
    ts = ts_ref[:]
    te = te_ref[:]
    p_ref[:] = (jnp.dot(ts, ass_ref[:], preferred_element_type=jnp.float32)
                + jnp.dot(te, aes_ref[:], preferred_element_type=jnp.float32)
                + bp_ref[:])
    q_ref[:] = (jnp.dot(ts, ase_ref[:], preferred_element_type=jnp.float32)
                + jnp.dot(te, aee_ref[:], preferred_element_type=jnp.float32)
                + bq_ref[:])


def _final_body(p_ref, q_ref, ts_ref, te_ref, mlc_ref, mlr_ref, o_ref, *, fb, kpad):
    i = pl.program_id(0)
    nt = (((1,), (1,)), ((), ()))
    c = (lax.dot_general(p_ref[:], ts_ref[:], nt, preferred_element_type=jnp.float32)
         + lax.dot_general(q_ref[:], te_ref[:], nt, preferred_element_type=jnp.float32))
    x = c + mlc_ref[:] + mlr_ref[:]
    rg = i * fb + lax.broadcasted_iota(jnp.int32, (fb, kpad), 0)
    cg = lax.broadcasted_iota(jnp.int32, (fb, kpad), 1)
    o_ref[:] = jnp.where(cg < rg, jnp.clip(x, -1e4, 1e4),
                         jnp.clip(x - 1e4, -1e4, 1e4))


def kernel(hidden_states, params, attention_mask):
    p = params
    x = hidden_states[0]
    s, h = x.shape
    f = p['sm_W'].shape[1]
    maxk = int(s * _TOP_LAMBDA)
    kpad = ((maxk + 31) // 32) * 32
    rb = 256 if s % 256 == 0 else s
    nrb = s // rb

    row = lambda v: v.reshape(1, -1)

    def mlp(pre):
        return pl.pallas_call(
            _mlp_body,
            grid=(nrb,),
            in_specs=[
                pl.BlockSpec((rb, h), lambda i: (i, 0)),
                pl.BlockSpec((h, f), lambda i: (0, 0)),
                pl.BlockSpec((1, f), lambda i: (0, 0)),
                pl.BlockSpec((1, f), lambda i: (0, 0)),
                pl.BlockSpec((1, f), lambda i: (0, 0)),
            ],
            out_specs=pl.BlockSpec((rb, f), lambda i: (i, 0)),
            out_shape=jax.ShapeDtypeStruct((s, f), jnp.float32),
        )(x, p[pre + '_W'], row(p[pre + '_b']), row(p[pre + '_g']),
          row(p[pre + '_beta']))

    sm = mlp('sm')
    em = mlp('em')
    scv = mlp('sc')
    ecv = mlp('ec')

    last = nrb - 1
    band = pl.pallas_call(
        functools.partial(_band_body, rb=rb, s=s),
        grid=(nrb,),
        in_specs=[
            pl.BlockSpec((rb, f), lambda i: (i, 0)),
            pl.BlockSpec((rb, f), lambda i: (i, 0)),
            pl.BlockSpec((rb, f), lambda i: (jnp.minimum(i + 1, last), 0)),
            pl.BlockSpec((f, f), lambda i: (0, 0)),
            pl.BlockSpec((1, f), lambda i: (0, 0)),
            pl.BlockSpec((f, 1), lambda i: (0, 0)),
            pl.BlockSpec((1, f), lambda i: (0, 0)),
        ],
        out_specs=pl.BlockSpec((rb, _BAND), lambda i: (i, 0)),
        out_shape=jax.ShapeDtypeStruct((s, _BAND), jnp.float32),
    )(sm, em, em, p['s2e_W'], row(p['s2e_b']), p['ms_W'], row(p['me_W'][:, 0]))

    sel_out = pl.pallas_call(
        functools.partial(_select_body, s=s, maxk=maxk, kpad=kpad),
        in_specs=[pl.BlockSpec((s, _BAND), lambda: (0, 0))],
        out_specs=pl.BlockSpec((kpad, 2), lambda: (0, 0)),
        out_shape=jax.ShapeDtypeStruct((kpad, 2), jnp.float32),
    )(band)

    code = jnp.round(sel_out[:, 0])
    ci = jnp.clip(code, 0.0, float(s * _BAND - 1)).astype(jnp.int32)
    sidx = jnp.clip(ci // _BAND, 0, s - 1)
    eidx = jnp.clip(sidx + (ci % _BAND), 0, s - 1)
    mlc = sel_out[:, 1:2]
    mlr = mlc.reshape(1, kpad)

    ts, te = pl.kernel(
        functools.partial(_sc_gather_body, nw2=kpad // 32),
        out_type=[jax.ShapeDtypeStruct((kpad, f), jnp.float32),
                  jax.ShapeDtypeStruct((kpad, f), jnp.float32)],
        mesh=plsc.VectorSubcoreMesh(core_axis_name="c", subcore_axis_name="s"),
        scratch_types=[
            pltpu.VMEM((16,), jnp.int32),
            pltpu.VMEM((16,), jnp.int32),
            pltpu.VMEM((16,), jnp.int32),
            pltpu.VMEM((16,), jnp.int32),
            pltpu.VMEM((16, f), jnp.float32),
            pltpu.SemaphoreType.DMA,
        ],
    )(sidx, eidx, scv, ecv)

    cb = 256 if f % 256 == 0 else f
    ncb = f // cb
    bp = row(p['a_s2s_b'] + p['a_e2s_b'])
    bq = row(p['a_s2e_b'] + p['a_e2e_b'])
    pm, qm = pl.pallas_call(
        _pq_body,
        grid=(ncb,),
        in_specs=[
            pl.BlockSpec((kpad, f), lambda j: (0, 0)),
            pl.BlockSpec((kpad, f), lambda j: (0, 0)),
            pl.BlockSpec((f, cb), lambda j: (0, j)),
            pl.BlockSpec((f, cb), lambda j: (0, j)),
            pl.BlockSpec((f, cb), lambda j: (0, j)),
            pl.BlockSpec((f, cb), lambda j: (0, j)),
            pl.BlockSpec((1, cb), lambda j: (0, j)),
            pl.BlockSpec((1, cb), lambda j: (0, j)),
        ],
        out_specs=[
            pl.BlockSpec((kpad, cb), lambda j: (0, j)),
            pl.BlockSpec((kpad, cb), lambda j: (0, j)),
        ],
        out_shape=[
            jax.ShapeDtypeStruct((kpad, f), jnp.float32),
            jax.ShapeDtypeStruct((kpad, f), jnp.float32),
        ],
    )(ts, te, p['a_s2s_W'], p['a_e2s_W'], p['a_s2e_W'], p['a_e2e_W'], bp, bq)

    fb = kpad // 4 if (kpad // 4) % 8 == 0 else kpad
    nfb = kpad // fb
    out = pl.pallas_call(
        functools.partial(_final_body, fb=fb, kpad=kpad),
        grid=(nfb,),
        in_specs=[
            pl.BlockSpec((fb, f), lambda i: (i, 0)),
            pl.BlockSpec((fb, f), lambda i: (i, 0)),
            pl.BlockSpec((kpad, f), lambda i: (0, 0)),
            pl.BlockSpec((kpad, f), lambda i: (0, 0)),
            pl.BlockSpec((fb, 1), lambda i: (i, 0)),
            pl.BlockSpec((1, kpad), lambda i: (0, 0)),
        ],
        out_specs=pl.BlockSpec((fb, kpad), lambda i: (i, 0)),
        out_shape=jax.ShapeDtypeStruct((kpad, kpad), jnp.float32),
    )(pm, qm, ts, te, mlc, mlr)

    return out[None, :maxk, :maxk]
